# baseline (speedup 1.0000x reference)
import functools

import jax
import jax.numpy as jnp
from jax import lax
from jax.experimental import pallas as pl
from jax.experimental.pallas import tpu as pltpu
from jax.experimental.pallas import tpu_sc as plsc

N_EMB = 8192
DIM = 64
BETA = 0.25
BZ = 512
CE = 1024


def _argmin_body(z_ref, emb_ref, idx_ref, loss_ref):
    zb = z_ref[...]
    znorm = jnp.sum(zb * zb, axis=1)
    run_min = jnp.full((BZ,), jnp.inf, jnp.float32)
    run_idx = jnp.zeros((BZ,), jnp.int32)
    for c in range(N_EMB // CE):
        eb = emb_ref[pl.ds(c * CE, CE), :]
        enorm = jnp.sum(eb * eb, axis=1)
        mm = lax.dot_general(zb, eb, (((1,), (1,)), ((), ())),
                             preferred_element_type=jnp.float32)
        t1 = znorm[:, None] + enorm[None, :]
        dists = t1 - 2.0 * mm
        cmin = jnp.min(dists, axis=1)
        match = dists == cmin[:, None]
        colidx = lax.broadcasted_iota(jnp.int32, (BZ, CE), 1)
        cidx = jnp.min(jnp.where(match, colidx, N_EMB), axis=1)
        upd = cmin < run_min
        run_idx = jnp.where(upd, c * CE + cidx, run_idx)
        run_min = jnp.where(upd, cmin, run_min)
    idx_ref[...] = run_idx

    @pl.when(pl.program_id(0) == 0)
    def _init():
        loss_ref[...] = jnp.zeros((1, 128), jnp.float32)

    loss_ref[...] += jnp.sum(run_min.reshape(-1, 128), axis=0, keepdims=True)


def _argmin_call(z, emb):
    nz = z.shape[0] // BZ
    return pl.pallas_call(
        _argmin_body,
        grid=(nz,),
        in_specs=[pl.BlockSpec((BZ, DIM), lambda i: (i, 0)),
                  pl.BlockSpec((N_EMB, DIM), lambda i: (0, 0))],
        out_specs=[pl.BlockSpec((BZ,), lambda i: (i,)),
                   pl.BlockSpec((1, 128), lambda i: (0, 0))],
        out_shape=[jax.ShapeDtypeStruct((z.shape[0],), jnp.int32),
                   jax.ShapeDtypeStruct((1, 128), jnp.float32)],
    )(z, emb)


def _gather_call(emb, idx):
    B = idx.shape[0]
    info = plsc.get_sparse_core_info()
    nw = info.num_cores * info.num_subcores
    b_per_w = B // nw
    mesh = plsc.VectorSubcoreMesh(core_axis_name="c", subcore_axis_name="s")

    @functools.partial(
        pl.kernel, mesh=mesh,
        compiler_params=pltpu.CompilerParams(use_tc_tiling_on_sc=False),
        out_type=jax.ShapeDtypeStruct((B, DIM), jnp.float32),
        scratch_types=[
            pltpu.VMEM((b_per_w,), jnp.int32),
            pltpu.VMEM((b_per_w, DIM), jnp.float32),
            pltpu.SemaphoreType.DMA,
        ],
    )
    def gather_k(table_hbm, idx_hbm, out_hbm, idx_v, rows_v, sem):
        wid = lax.axis_index("s") * info.num_cores + lax.axis_index("c")
        base = wid * b_per_w
        pltpu.sync_copy(idx_hbm.at[pl.ds(base, b_per_w)], idx_v)
        pltpu.async_copy(table_hbm.at[idx_v], rows_v, sem).wait()
        pltpu.sync_copy(rows_v, out_hbm.at[pl.ds(base, b_per_w)])

    return gather_k(emb, idx)


def kernel(z, emb):
    idx, loss_acc = _argmin_call(z, emb)
    z_q = _gather_call(emb, idx)
    loss = BETA * (jnp.sum(loss_acc) / (z.shape[0] * DIM))
    z_q_st = z + (z_q - z)
    return (z_q_st, loss, loss, idx)


# speedup vs baseline: 1.6351x; 1.6351x over previous
import functools

import jax
import jax.numpy as jnp
from jax import lax
from jax.experimental import pallas as pl
from jax.experimental.pallas import tpu as pltpu
from jax.experimental.pallas import tpu_sc as plsc

N_EMB = 8192
DIM = 64
BETA = 0.25
BZ = 512
CE = 1024


def _argmin_body(z_ref, emb_ref, idx_ref, loss_ref):
    zb = z_ref[...]
    zb2 = zb + zb
    znorm = jnp.sum(zb * zb, axis=1)
    m = jnp.full((BZ, 128), jnp.inf, jnp.float32)
    a = jnp.zeros((BZ, 128), jnp.int32)
    nvc = CE // 128
    for c in range(N_EMB // CE):
        eb = emb_ref[pl.ds(c * CE, CE), :]
        enorm = jnp.sum(eb * eb, axis=1)
        mm2 = lax.dot_general(zb2, eb, (((1,), (1,)), ((), ())),
                              preferred_element_type=jnp.float32)
        for v in range(nvc):
            sl = slice(v * 128, (v + 1) * 128)
            t1 = znorm[:, None] + enorm[None, sl]
            d = t1 - mm2[:, sl]
            upd = d < m
            m = jnp.where(upd, d, m)
            a = jnp.where(upd, c * nvc + v, a)
    cmin = jnp.min(m, axis=1)
    lanes = lax.broadcasted_iota(jnp.int32, (BZ, 128), 1)
    packed = a * 128 + lanes
    run_idx = jnp.min(jnp.where(m == cmin[:, None], packed, N_EMB), axis=1)
    run_min = cmin
    idx_ref[...] = run_idx

    @pl.when(pl.program_id(0) == 0)
    def _init():
        loss_ref[...] = jnp.zeros((1, 128), jnp.float32)

    loss_ref[...] += jnp.sum(run_min.reshape(-1, 128), axis=0, keepdims=True)


def _argmin_call(z, emb):
    nz = z.shape[0] // BZ
    return pl.pallas_call(
        _argmin_body,
        grid=(nz,),
        in_specs=[pl.BlockSpec((BZ, DIM), lambda i: (i, 0)),
                  pl.BlockSpec((N_EMB, DIM), lambda i: (0, 0))],
        out_specs=[pl.BlockSpec((BZ,), lambda i: (i,)),
                   pl.BlockSpec((1, 128), lambda i: (0, 0))],
        out_shape=[jax.ShapeDtypeStruct((z.shape[0],), jnp.int32),
                   jax.ShapeDtypeStruct((1, 128), jnp.float32)],
    )(z, emb)


def _gather_call(emb, idx):
    B = idx.shape[0]
    info = plsc.get_sparse_core_info()
    nw = info.num_cores * info.num_subcores
    b_per_w = B // nw
    mesh = plsc.VectorSubcoreMesh(core_axis_name="c", subcore_axis_name="s")

    @functools.partial(
        pl.kernel, mesh=mesh,
        compiler_params=pltpu.CompilerParams(use_tc_tiling_on_sc=False),
        out_type=jax.ShapeDtypeStruct((B, DIM), jnp.float32),
        scratch_types=[
            pltpu.VMEM((b_per_w,), jnp.int32),
            pltpu.VMEM((b_per_w, DIM), jnp.float32),
            pltpu.SemaphoreType.DMA,
        ],
    )
    def gather_k(table_hbm, idx_hbm, out_hbm, idx_v, rows_v, sem):
        wid = lax.axis_index("s") * info.num_cores + lax.axis_index("c")
        base = wid * b_per_w
        pltpu.sync_copy(idx_hbm.at[pl.ds(base, b_per_w)], idx_v)
        pltpu.async_copy(table_hbm.at[idx_v], rows_v, sem).wait()
        pltpu.sync_copy(rows_v, out_hbm.at[pl.ds(base, b_per_w)])

    return gather_k(emb, idx)


def kernel(z, emb):
    idx, loss_acc = _argmin_call(z, emb)
    z_q = _gather_call(emb, idx)
    loss = BETA * (jnp.sum(loss_acc) / (z.shape[0] * DIM))
    z_q_st = z + (z_q - z)
    return (z_q_st, loss, loss, idx)
